# Initial kernel scaffold; baseline (speedup 1.0000x reference)
import functools

import jax
import jax.numpy as jnp
from jax import lax
from jax.experimental import pallas as pl
from jax.experimental.pallas import tpu as pltpu
from jax.experimental.pallas import tpu_sc as plsc

_NC = 2
_NS = 16
_L = 16
_NW = _NC * _NS

_MESH = dict(core_axis_name="c", subcore_axis_name="s")


def _degree_partials(cols3, n_pad):
    nw, nchunk, ch = cols3.shape
    zps = n_pad // _NS

    @functools.partial(
        pl.kernel,
        out_type=jax.ShapeDtypeStruct((_NC, n_pad), jnp.float32),
        mesh=plsc.VectorSubcoreMesh(**_MESH),
        scratch_types=[
            pltpu.VMEM((nchunk, ch), jnp.int32),
            pltpu.VMEM((ch,), jnp.float32),
            pltpu.VMEM((zps,), jnp.float32),
            pltpu.VMEM_SHARED((n_pad,), jnp.float32),
            pltpu.SemaphoreType.DMA,
        ],
    )
    def deg_kernel(cols_hbm, out_hbm, cidx, ones_v, zeros_v, deg_sh, sem):
        cid = lax.axis_index("c")
        sid = lax.axis_index("s")
        wid = cid * _NS + sid
        pltpu.async_copy(cols_hbm.at[wid], cidx, sem).wait()

        @pl.loop(0, ch, step=_L)
        def _(i):
            ones_v[pl.ds(i, _L)] = jnp.ones((_L,), jnp.float32)

        @pl.loop(0, zps, step=_L)
        def _(i):
            zeros_v[pl.ds(i, _L)] = jnp.zeros((_L,), jnp.float32)

        pltpu.sync_copy(zeros_v, deg_sh.at[pl.ds(sid * zps, zps)])
        plsc.subcore_barrier()

        @pl.loop(0, nchunk)
        def _(j):
            pltpu.sync_copy(ones_v, deg_sh.at[cidx.at[j]], add=True)

        plsc.subcore_barrier()
        pltpu.sync_copy(deg_sh.at[pl.ds(sid * zps, zps)],
                        out_hbm.at[cid, pl.ds(sid * zps, zps)])

    return deg_kernel(cols3)


def _aggregate(y, rows3, cols3):
    n, d = y.shape
    nw, nchunk, ch = rows3.shape
    npc = n // _NS
    zr = 125

    @functools.partial(
        pl.kernel,
        out_type=jax.ShapeDtypeStruct((_NC, n, d), jnp.float32),
        mesh=plsc.VectorSubcoreMesh(**_MESH),
        scratch_types=[
            pltpu.VMEM((nchunk, ch), jnp.int32),
            pltpu.VMEM((nchunk, ch), jnp.int32),
            pltpu.VMEM((ch, d), jnp.float32),
            pltpu.VMEM((zr, d), jnp.float32),
            pltpu.VMEM_SHARED((n, d), jnp.float32),
            pltpu.SemaphoreType.DMA,
        ],
    )
    def agg_kernel(y_hbm, rows_hbm, cols_hbm, out_hbm,
                   ridx, cidx, buf, zeros_v, agg_sh, sem):
        cid = lax.axis_index("c")
        sid = lax.axis_index("s")
        wid = cid * _NS + sid
        pltpu.async_copy(rows_hbm.at[wid], ridx, sem).wait()
        pltpu.async_copy(cols_hbm.at[wid], cidx, sem).wait()

        @pl.loop(0, zr)
        def _(r):
            @pl.loop(0, d, step=_L)
            def _(c0):
                zeros_v[r, pl.ds(c0, _L)] = jnp.zeros((_L,), jnp.float32)

        @pl.loop(0, npc, step=zr)
        def _(r0):
            pltpu.sync_copy(zeros_v, agg_sh.at[pl.ds(sid * npc + r0, zr)])

        plsc.subcore_barrier()

        @pl.loop(0, nchunk)
        def _(j):
            pltpu.async_copy(y_hbm.at[ridx.at[j]], buf, sem).wait()
            pltpu.sync_copy(buf, agg_sh.at[cidx.at[j]], add=True)

        plsc.subcore_barrier()

        @pl.loop(0, npc, step=zr)
        def _(r0):
            pltpu.sync_copy(agg_sh.at[pl.ds(sid * npc + r0, zr)],
                            out_hbm.at[cid, pl.ds(sid * npc + r0, zr)])

    return agg_kernel(y, rows3, cols3)


_BN = 2000


def _mm_body(x_ref, w_ref, o_ref):
    o_ref[...] = jnp.dot(x_ref[...], w_ref[...],
                         preferred_element_type=jnp.float32)


def _matmul(x, w):
    n, din = x.shape
    dout = w.shape[1]
    return pl.pallas_call(
        _mm_body,
        grid=(n // _BN,),
        in_specs=[pl.BlockSpec((_BN, din), lambda i: (i, 0)),
                  pl.BlockSpec((din, dout), lambda i: (0, 0))],
        out_specs=pl.BlockSpec((_BN, dout), lambda i: (i, 0)),
        out_shape=jax.ShapeDtypeStruct((n, dout), jnp.float32),
    )(x, w)


def _scale_body(t_ref, dp_ref, y_ref, dis_ref):
    deg = dp_ref[0] + dp_ref[1] + 1.0
    dis = lax.rsqrt(deg)
    y_ref[...] = t_ref[...] * dis[:, None]
    dis_ref[...] = dis


def _scale(t, dp):
    n, d = t.shape
    return pl.pallas_call(
        _scale_body,
        grid=(n // _BN,),
        in_specs=[pl.BlockSpec((_BN, d), lambda i: (i, 0)),
                  pl.BlockSpec((2, _BN), lambda i: (0, i))],
        out_specs=[pl.BlockSpec((_BN, d), lambda i: (i, 0)),
                   pl.BlockSpec((_BN,), lambda i: (i,))],
        out_shape=[jax.ShapeDtypeStruct((n, d), jnp.float32),
                   jax.ShapeDtypeStruct((n,), jnp.float32)],
    )(t, dp)


def _mid_body(y_ref, p0_ref, p1_ref, dis_ref, b_ref, w_ref, o_ref):
    dis = dis_ref[...]
    h = (y_ref[...] + p0_ref[...] + p1_ref[...]) * dis[:, None] + b_ref[...]
    o_ref[...] = jnp.dot(h, w_ref[...],
                         preferred_element_type=jnp.float32) * dis[:, None]


def _mid(y, p0, p1, dis, b, w):
    n, d = y.shape
    dout = w.shape[1]
    blk2 = pl.BlockSpec((_BN, d), lambda i: (i, 0))
    return pl.pallas_call(
        _mid_body,
        grid=(n // _BN,),
        in_specs=[blk2, blk2, blk2,
                  pl.BlockSpec((_BN,), lambda i: (i,)),
                  pl.BlockSpec((1, d), lambda i: (0, 0)),
                  pl.BlockSpec((d, dout), lambda i: (0, 0))],
        out_specs=pl.BlockSpec((_BN, dout), lambda i: (i, 0)),
        out_shape=jax.ShapeDtypeStruct((n, dout), jnp.float32),
    )(y, p0, p1, dis, b, w)


def _final_body(y_ref, q0_ref, q1_ref, dis_ref, b_ref, o_ref):
    dis = dis_ref[...]
    o_ref[...] = (y_ref[...] + q0_ref[...] + q1_ref[...]) * dis[:, None] \
        + b_ref[...]


def _final(y, q0, q1, dis, b):
    n, d = y.shape
    blk2 = pl.BlockSpec((_BN, d), lambda i: (i, 0))
    return pl.pallas_call(
        _final_body,
        grid=(n // _BN,),
        in_specs=[blk2, blk2, blk2,
                  pl.BlockSpec((_BN,), lambda i: (i,)),
                  pl.BlockSpec((1, d), lambda i: (0, 0))],
        out_specs=pl.BlockSpec((_BN, d), lambda i: (i, 0)),
        out_shape=jax.ShapeDtypeStruct((n, d), jnp.float32),
    )(y, q0, q1, dis, b)


def kernel(x, edge_index, W1, b1, W2, b2):
    n, _ = x.shape
    e = edge_index.shape[1]
    epw = e // _NW
    ch = 80
    nchunk = epw // ch
    rows3 = edge_index[0].reshape(_NW, nchunk, ch)
    cols3 = edge_index[1].reshape(_NW, nchunk, ch)
    n_pad = -(-n // (_NS * 8)) * (_NS * 8)

    degp = _degree_partials(cols3, n_pad)
    t1 = _matmul(x, W1)
    y1, dis = _scale(t1, degp[:, :n])
    p = _aggregate(y1, rows3, cols3)
    y2 = _mid(y1, p[0], p[1], dis, b1.reshape(1, -1), W2)
    q = _aggregate(y2, rows3, cols3)
    return _final(y2, q[0], q[1], dis, b2.reshape(1, -1))


# speedup vs baseline: 20.1114x; 20.1114x over previous
import functools

import jax
import jax.numpy as jnp
from jax import lax
from jax.experimental import pallas as pl
from jax.experimental.pallas import tpu as pltpu
from jax.experimental.pallas import tpu_sc as plsc

_NC = 2
_NS = 16
_L = 16
_NW = _NC * _NS

_MESH = dict(core_axis_name="c", subcore_axis_name="s")


def _degree_partials(cols3, n_pad):
    nw, nchunk, ch = cols3.shape
    zps = n_pad // _NS

    @functools.partial(
        pl.kernel,
        out_type=jax.ShapeDtypeStruct((_NC * n_pad,), jnp.float32),
        mesh=plsc.VectorSubcoreMesh(**_MESH),
        scratch_types=[
            pltpu.VMEM((nchunk, ch), jnp.int32),
            pltpu.VMEM((ch,), jnp.float32),
            pltpu.VMEM((zps,), jnp.float32),
            pltpu.VMEM_SHARED((n_pad,), jnp.float32),
            pltpu.SemaphoreType.DMA,
        ],
    )
    def deg_kernel(cols_hbm, out_hbm, cidx, ones_v, zeros_v, deg_sh, sem):
        cid = lax.axis_index("c")
        sid = lax.axis_index("s")
        wid = cid * _NS + sid
        pltpu.async_copy(cols_hbm.at[wid], cidx, sem).wait()

        @pl.loop(0, ch, step=_L)
        def _(i):
            ones_v[pl.ds(i, _L)] = jnp.ones((_L,), jnp.float32)

        @pl.loop(0, zps, step=_L)
        def _(i):
            zeros_v[pl.ds(i, _L)] = jnp.zeros((_L,), jnp.float32)

        pltpu.sync_copy(zeros_v, deg_sh.at[pl.ds(sid * zps, zps)])
        plsc.subcore_barrier()

        @pl.loop(0, nchunk)
        def _(j):
            pltpu.sync_copy(ones_v, deg_sh.at[cidx.at[j]], add=True)

        plsc.subcore_barrier()
        pltpu.sync_copy(deg_sh.at[pl.ds(sid * zps, zps)], zeros_v)
        pltpu.sync_copy(zeros_v, out_hbm.at[pl.ds(cid * n_pad + sid * zps, zps)])

    return deg_kernel(cols3)


def _aggregate(y, rows3, cols3, n_acc):
    n, d = y.shape
    nw, nchunk, ch = rows3.shape
    npc = n_acc // _NS

    @functools.partial(
        pl.kernel,
        out_type=jax.ShapeDtypeStruct((_NC, n_acc, d), jnp.float32),
        mesh=plsc.VectorSubcoreMesh(**_MESH),
        scratch_types=[
            pltpu.VMEM((nchunk, ch), jnp.int32),
            pltpu.VMEM((nchunk, ch), jnp.int32),
            pltpu.VMEM((ch, d), jnp.float32),
            pltpu.VMEM_SHARED((n_acc, d), jnp.float32),
            pltpu.SemaphoreType.DMA,
        ],
    )
    def agg_kernel(y_hbm, rows_hbm, cols_hbm, out_hbm,
                   ridx, cidx, buf, agg_sh, sem):
        cid = lax.axis_index("c")
        sid = lax.axis_index("s")
        wid = cid * _NS + sid
        pltpu.async_copy(rows_hbm.at[wid], ridx, sem).wait()
        pltpu.async_copy(cols_hbm.at[wid], cidx, sem).wait()

        @pl.loop(0, ch)
        def _(r):
            @pl.loop(0, d, step=_L)
            def _(c0):
                buf[r, pl.ds(c0, _L)] = jnp.zeros((_L,), jnp.float32)

        @pl.loop(0, npc, step=ch)
        def _(r0):
            pltpu.sync_copy(buf, agg_sh.at[pl.ds(sid * npc + r0, ch)])

        plsc.subcore_barrier()

        @pl.loop(0, nchunk)
        def _(j):
            pltpu.async_copy(y_hbm.at[ridx.at[j]], buf, sem).wait()
            pltpu.sync_copy(buf, agg_sh.at[cidx.at[j]], add=True)

        plsc.subcore_barrier()

        @pl.loop(0, npc, step=ch)
        def _(r0):
            pltpu.sync_copy(agg_sh.at[pl.ds(sid * npc + r0, ch)], buf)
            pltpu.sync_copy(buf,
                            out_hbm.at[cid, pl.ds(sid * npc + r0, ch)])

    return agg_kernel(y, rows3, cols3)


_BN = 2000


def _mm_body(x_ref, w_ref, o_ref):
    o_ref[...] = jnp.dot(x_ref[...], w_ref[...],
                         preferred_element_type=jnp.float32)


def _matmul(x, w):
    n, din = x.shape
    dout = w.shape[1]
    return pl.pallas_call(
        _mm_body,
        grid=(n // _BN,),
        in_specs=[pl.BlockSpec((_BN, din), lambda i: (i, 0)),
                  pl.BlockSpec((din, dout), lambda i: (0, 0))],
        out_specs=pl.BlockSpec((_BN, dout), lambda i: (i, 0)),
        out_shape=jax.ShapeDtypeStruct((n, dout), jnp.float32),
    )(x, w)


def _scale_body(t_ref, d0_ref, d1_ref, y_ref, dis_ref):
    deg = d0_ref[...] + d1_ref[...] + 1.0
    dis = lax.rsqrt(deg)
    y_ref[...] = t_ref[...] * dis
    dis_ref[...] = dis


def _scale(t, d0, d1):
    n, d = t.shape
    blk1 = pl.BlockSpec((_BN, 1), lambda i: (i, 0))
    return pl.pallas_call(
        _scale_body,
        grid=(n // _BN,),
        in_specs=[pl.BlockSpec((_BN, d), lambda i: (i, 0)), blk1, blk1],
        out_specs=[pl.BlockSpec((_BN, d), lambda i: (i, 0)), blk1],
        out_shape=[jax.ShapeDtypeStruct((n, d), jnp.float32),
                   jax.ShapeDtypeStruct((n, 1), jnp.float32)],
    )(t, d0, d1)


def _mid_body(y_ref, p0_ref, p1_ref, dis_ref, b_ref, w_ref, o_ref):
    dis = dis_ref[...]
    h = (y_ref[...] + p0_ref[0] + p1_ref[0]) * dis + b_ref[...]
    o_ref[...] = jnp.dot(h, w_ref[...],
                         preferred_element_type=jnp.float32) * dis


def _mid(y, p, dis, b, w):
    n, d = y.shape
    dout = w.shape[1]
    blk2 = pl.BlockSpec((_BN, d), lambda i: (i, 0))
    return pl.pallas_call(
        _mid_body,
        grid=(n // _BN,),
        in_specs=[blk2,
                  pl.BlockSpec((1, _BN, d), lambda i: (0, i, 0)),
                  pl.BlockSpec((1, _BN, d), lambda i: (1, i, 0)),
                  pl.BlockSpec((_BN, 1), lambda i: (i, 0)),
                  pl.BlockSpec((1, d), lambda i: (0, 0)),
                  pl.BlockSpec((d, dout), lambda i: (0, 0))],
        out_specs=pl.BlockSpec((_BN, dout), lambda i: (i, 0)),
        out_shape=jax.ShapeDtypeStruct((n, dout), jnp.float32),
    )(y, p, p, dis, b, w)


def _final_body(y_ref, q0_ref, q1_ref, dis_ref, b_ref, o_ref):
    o_ref[...] = (y_ref[...] + q0_ref[0] + q1_ref[0]) * dis_ref[...] \
        + b_ref[...]


def _final(y, q, dis, b):
    n, d = y.shape
    blk2 = pl.BlockSpec((_BN, d), lambda i: (i, 0))
    return pl.pallas_call(
        _final_body,
        grid=(n // _BN,),
        in_specs=[blk2,
                  pl.BlockSpec((1, _BN, d), lambda i: (0, i, 0)),
                  pl.BlockSpec((1, _BN, d), lambda i: (1, i, 0)),
                  pl.BlockSpec((_BN, 1), lambda i: (i, 0)),
                  pl.BlockSpec((1, d), lambda i: (0, 0))],
        out_specs=pl.BlockSpec((_BN, d), lambda i: (i, 0)),
        out_shape=jax.ShapeDtypeStruct((n, d), jnp.float32),
    )(y, q, q, dis, b)


def kernel(x, edge_index, W1, b1, W2, b2):
    n, _ = x.shape
    e = edge_index.shape[1]
    epw = e // _NW
    ch = 80
    nchunk = epw // ch
    rows3 = edge_index[0].reshape(_NW, nchunk, ch)
    cols3 = edge_index[1].reshape(_NW, nchunk, ch)
    n_pad = -(-n // (_NS * 8)) * (_NS * 8)
    n_acc = -(-n // (_NS * 128)) * (_NS * 128)

    degp = _degree_partials(cols3, n_pad).reshape(_NC, n_pad)
    t1 = _matmul(x, W1)
    d0 = degp[0, :n].reshape(n, 1)
    d1 = degp[1, :n].reshape(n, 1)
    y1, dis = _scale(t1, d0, d1)
    p = _aggregate(y1, rows3, cols3, n_acc)
    y2 = _mid(y1, p, dis, b1.reshape(1, -1), W2)
    q = _aggregate(y2, rows3, cols3, n_acc)
    return _final(y2, q, dis, b2.reshape(1, -1))


# speedup vs baseline: 24.7198x; 1.2291x over previous
import functools

import jax
import jax.numpy as jnp
from jax import lax
from jax.experimental import pallas as pl
from jax.experimental.pallas import tpu as pltpu
from jax.experimental.pallas import tpu_sc as plsc

_NC = 2
_NS = 16
_L = 16
_NW = _NC * _NS

_MESH = dict(core_axis_name="c", subcore_axis_name="s")


def _degree_partials(cols4, n_pad):
    nw, nwin, wchunk, ch = cols4.shape
    zps = n_pad // _NS

    @functools.partial(
        pl.kernel,
        out_type=jax.ShapeDtypeStruct((_NC * n_pad,), jnp.float32),
        mesh=plsc.VectorSubcoreMesh(**_MESH),
        scratch_types=[
            pltpu.VMEM((wchunk, ch), jnp.int32),
            pltpu.VMEM((ch,), jnp.float32),
            pltpu.VMEM((zps,), jnp.float32),
            pltpu.VMEM_SHARED((n_pad,), jnp.float32),
            pltpu.SemaphoreType.DMA,
        ],
    )
    def deg_kernel(cols_hbm, out_hbm, cidx, ones_v, zeros_v, deg_sh, sem):
        cid = lax.axis_index("c")
        sid = lax.axis_index("s")
        wid = cid * _NS + sid

        @pl.loop(0, ch, step=_L)
        def _(i):
            ones_v[pl.ds(i, _L)] = jnp.ones((_L,), jnp.float32)

        @pl.loop(0, zps, step=_L)
        def _(i):
            zeros_v[pl.ds(i, _L)] = jnp.zeros((_L,), jnp.float32)

        pltpu.sync_copy(zeros_v, deg_sh.at[pl.ds(sid * zps, zps)])
        plsc.subcore_barrier()

        @pl.loop(0, nwin)
        def _(w):
            pltpu.sync_copy(cols_hbm.at[wid, w], cidx)

            @pl.loop(0, wchunk)
            def _(j):
                pltpu.sync_copy(ones_v, deg_sh.at[cidx.at[j]], add=True)

        plsc.subcore_barrier()
        pltpu.sync_copy(deg_sh.at[pl.ds(sid * zps, zps)], zeros_v)
        pltpu.sync_copy(zeros_v, out_hbm.at[pl.ds(cid * n_pad + sid * zps, zps)])

    return deg_kernel(cols4)


def _aggregate(y, rows4, cols4, n_acc):
    n, d = y.shape
    nw, nwin, wchunk, ch = rows4.shape
    npc = n_acc // _NS

    @functools.partial(
        pl.kernel,
        out_type=jax.ShapeDtypeStruct((_NC, n_acc, d), jnp.float32),
        mesh=plsc.VectorSubcoreMesh(**_MESH),
        scratch_types=[
            pltpu.VMEM((wchunk, ch), jnp.int32),
            pltpu.VMEM((wchunk, ch), jnp.int32),
            pltpu.VMEM((ch, d), jnp.float32),
            pltpu.VMEM((ch, d), jnp.float32),
            pltpu.VMEM_SHARED((n_acc, d), jnp.float32),
            pltpu.SemaphoreType.DMA,
            pltpu.SemaphoreType.DMA,
            pltpu.SemaphoreType.DMA,
        ],
    )
    def agg_kernel(y_hbm, rows_hbm, cols_hbm, out_hbm,
                   ridx, cidx, bufa, bufb, agg_sh, sem, sema, semb):
        cid = lax.axis_index("c")
        sid = lax.axis_index("s")
        wid = cid * _NS + sid

        @pl.loop(0, ch)
        def _(r):
            @pl.loop(0, d, step=_L)
            def _(c0):
                bufa[r, pl.ds(c0, _L)] = jnp.zeros((_L,), jnp.float32)

        @pl.loop(0, npc, step=ch)
        def _(r0):
            pltpu.sync_copy(bufa, agg_sh.at[pl.ds(sid * npc + r0, ch)])

        plsc.subcore_barrier()

        @pl.loop(0, nwin)
        def _(w):
            pltpu.async_copy(rows_hbm.at[wid, w], ridx, sem)
            pltpu.async_copy(cols_hbm.at[wid, w], cidx, sem)
            pltpu.make_async_copy(rows_hbm.at[wid, w], ridx, sem).wait()
            pltpu.make_async_copy(cols_hbm.at[wid, w], cidx, sem).wait()
            pltpu.async_copy(y_hbm.at[ridx.at[0]], bufa, sema)

            @pl.loop(0, wchunk - 1, step=2)
            def _(j):
                pltpu.make_async_copy(y_hbm.at[ridx.at[j]], bufa, sema).wait()
                pltpu.async_copy(y_hbm.at[ridx.at[j + 1]], bufb, semb)
                pltpu.sync_copy(bufa, agg_sh.at[cidx.at[j]], add=True)
                pltpu.make_async_copy(y_hbm.at[ridx.at[j + 1]], bufb,
                                      semb).wait()
                pltpu.async_copy(y_hbm.at[ridx.at[j + 2]], bufa, sema)
                pltpu.sync_copy(bufb, agg_sh.at[cidx.at[j + 1]], add=True)

            pltpu.make_async_copy(y_hbm.at[ridx.at[wchunk - 1]], bufa,
                                  sema).wait()
            pltpu.sync_copy(bufa, agg_sh.at[cidx.at[wchunk - 1]], add=True)

        plsc.subcore_barrier()

        @pl.loop(0, npc, step=ch)
        def _(r0):
            pltpu.sync_copy(agg_sh.at[pl.ds(sid * npc + r0, ch)], bufa)
            pltpu.sync_copy(bufa,
                            out_hbm.at[cid, pl.ds(sid * npc + r0, ch)])

    return agg_kernel(y, rows4, cols4)


_BN = 2000


def _mm_body(x_ref, w_ref, o_ref):
    o_ref[...] = jnp.dot(x_ref[...], w_ref[...],
                         preferred_element_type=jnp.float32)


def _matmul(x, w):
    n, din = x.shape
    dout = w.shape[1]
    return pl.pallas_call(
        _mm_body,
        grid=(n // _BN,),
        in_specs=[pl.BlockSpec((_BN, din), lambda i: (i, 0)),
                  pl.BlockSpec((din, dout), lambda i: (0, 0))],
        out_specs=pl.BlockSpec((_BN, dout), lambda i: (i, 0)),
        out_shape=jax.ShapeDtypeStruct((n, dout), jnp.float32),
    )(x, w)


def _scale_body(t_ref, d0_ref, d1_ref, y_ref, dis_ref):
    deg = d0_ref[...] + d1_ref[...] + 1.0
    dis = lax.rsqrt(deg)
    y_ref[...] = t_ref[...] * dis
    dis_ref[...] = dis


def _scale(t, d0, d1):
    n, d = t.shape
    blk1 = pl.BlockSpec((_BN, 1), lambda i: (i, 0))
    return pl.pallas_call(
        _scale_body,
        grid=(n // _BN,),
        in_specs=[pl.BlockSpec((_BN, d), lambda i: (i, 0)), blk1, blk1],
        out_specs=[pl.BlockSpec((_BN, d), lambda i: (i, 0)), blk1],
        out_shape=[jax.ShapeDtypeStruct((n, d), jnp.float32),
                   jax.ShapeDtypeStruct((n, 1), jnp.float32)],
    )(t, d0, d1)


def _mid_body(y_ref, p0_ref, p1_ref, dis_ref, b_ref, w_ref, o_ref):
    dis = dis_ref[...]
    h = (y_ref[...] + p0_ref[0] + p1_ref[0]) * dis + b_ref[...]
    o_ref[...] = jnp.dot(h, w_ref[...],
                         preferred_element_type=jnp.float32) * dis


def _mid(y, p, dis, b, w):
    n, d = y.shape
    dout = w.shape[1]
    blk2 = pl.BlockSpec((_BN, d), lambda i: (i, 0))
    return pl.pallas_call(
        _mid_body,
        grid=(n // _BN,),
        in_specs=[blk2,
                  pl.BlockSpec((1, _BN, d), lambda i: (0, i, 0)),
                  pl.BlockSpec((1, _BN, d), lambda i: (1, i, 0)),
                  pl.BlockSpec((_BN, 1), lambda i: (i, 0)),
                  pl.BlockSpec((1, d), lambda i: (0, 0)),
                  pl.BlockSpec((d, dout), lambda i: (0, 0))],
        out_specs=pl.BlockSpec((_BN, dout), lambda i: (i, 0)),
        out_shape=jax.ShapeDtypeStruct((n, dout), jnp.float32),
    )(y, p, p, dis, b, w)


def _final_body(y_ref, q0_ref, q1_ref, dis_ref, b_ref, o_ref):
    o_ref[...] = (y_ref[...] + q0_ref[0] + q1_ref[0]) * dis_ref[...] \
        + b_ref[...]


def _final(y, q, dis, b):
    n, d = y.shape
    blk2 = pl.BlockSpec((_BN, d), lambda i: (i, 0))
    return pl.pallas_call(
        _final_body,
        grid=(n // _BN,),
        in_specs=[blk2,
                  pl.BlockSpec((1, _BN, d), lambda i: (0, i, 0)),
                  pl.BlockSpec((1, _BN, d), lambda i: (1, i, 0)),
                  pl.BlockSpec((_BN, 1), lambda i: (i, 0)),
                  pl.BlockSpec((1, d), lambda i: (0, 0))],
        out_specs=pl.BlockSpec((_BN, d), lambda i: (i, 0)),
        out_shape=jax.ShapeDtypeStruct((n, d), jnp.float32),
    )(y, q, q, dis, b)


def kernel(x, edge_index, W1, b1, W2, b2):
    n, _ = x.shape
    e = edge_index.shape[1]
    epw = e // _NW
    ch = 80
    nwin = 5
    wchunk = epw // ch // nwin
    rows4 = edge_index[0].reshape(_NW, nwin, wchunk, ch)
    cols4 = edge_index[1].reshape(_NW, nwin, wchunk, ch)
    n_pad = -(-n // (_NS * 8)) * (_NS * 8)
    n_acc = -(-n // (_NS * 128)) * (_NS * 128)

    degp = _degree_partials(cols4, n_pad).reshape(_NC, n_pad)
    t1 = _matmul(x, W1)
    d0 = degp[0, :n].reshape(n, 1)
    d1 = degp[1, :n].reshape(n, 1)
    y1, dis = _scale(t1, d0, d1)
    p = _aggregate(y1, rows4, cols4, n_acc)
    y2 = _mid(y1, p, dis, b1.reshape(1, -1), W2)
    q = _aggregate(y2, rows4, cols4, n_acc)
    return _final(y2, q, dis, b2.reshape(1, -1))


# speedup vs baseline: 32.6643x; 1.3214x over previous
import functools

import jax
import jax.numpy as jnp
from jax import lax
from jax.experimental import pallas as pl
from jax.experimental.pallas import tpu as pltpu
from jax.experimental.pallas import tpu_sc as plsc

_NC = 2
_NS = 16
_L = 16
_NW = _NC * _NS

_MESH = dict(core_axis_name="c", subcore_axis_name="s")


def _degree_partials(cols4, n_pad):
    nw, nwin, wchunk, ch = cols4.shape
    zps = n_pad // _NS

    @functools.partial(
        pl.kernel,
        out_type=jax.ShapeDtypeStruct((_NC * n_pad,), jnp.float32),
        mesh=plsc.VectorSubcoreMesh(**_MESH),
        scratch_types=[
            pltpu.VMEM((wchunk, ch), jnp.int32),
            pltpu.VMEM((ch,), jnp.float32),
            pltpu.VMEM((zps,), jnp.float32),
            pltpu.VMEM_SHARED((n_pad,), jnp.float32),
            pltpu.SemaphoreType.DMA,
        ],
    )
    def deg_kernel(cols_hbm, out_hbm, cidx, ones_v, zeros_v, deg_sh, sem):
        cid = lax.axis_index("c")
        sid = lax.axis_index("s")
        wid = cid * _NS + sid

        @pl.loop(0, ch, step=_L)
        def _(i):
            ones_v[pl.ds(i, _L)] = jnp.ones((_L,), jnp.float32)

        @pl.loop(0, zps, step=_L)
        def _(i):
            zeros_v[pl.ds(i, _L)] = jnp.zeros((_L,), jnp.float32)

        pltpu.sync_copy(zeros_v, deg_sh.at[pl.ds(sid * zps, zps)])
        plsc.subcore_barrier()

        @pl.loop(0, nwin)
        def _(w):
            pltpu.sync_copy(cols_hbm.at[wid, w], cidx)

            @pl.loop(0, wchunk)
            def _(j):
                pltpu.sync_copy(ones_v, deg_sh.at[cidx.at[j]], add=True)

        plsc.subcore_barrier()
        pltpu.sync_copy(deg_sh.at[pl.ds(sid * zps, zps)], zeros_v)
        pltpu.sync_copy(zeros_v, out_hbm.at[pl.ds(cid * n_pad + sid * zps, zps)])

    return deg_kernel(cols4)


def _aggregate(y, rows4, cols4, n_acc):
    n, d = y.shape
    nw, nwin, wchunk, ch = rows4.shape
    npc = n_acc // _NS
    nring = 4
    rem = wchunk % nring
    main_hi = wchunk - nring - rem

    @functools.partial(
        pl.kernel,
        out_type=jax.ShapeDtypeStruct((_NC, n_acc, d), jnp.float32),
        mesh=plsc.VectorSubcoreMesh(**_MESH),
        scratch_types=[
            pltpu.VMEM((wchunk, ch), jnp.int32),
            pltpu.VMEM((wchunk, ch), jnp.int32),
            pltpu.VMEM((ch, d), jnp.float32),
            pltpu.VMEM((ch, d), jnp.float32),
            pltpu.VMEM((ch, d), jnp.float32),
            pltpu.VMEM((ch, d), jnp.float32),
            pltpu.VMEM_SHARED((n_acc, d), jnp.float32),
            pltpu.SemaphoreType.DMA,
            pltpu.SemaphoreType.DMA,
            pltpu.SemaphoreType.DMA,
            pltpu.SemaphoreType.DMA,
            pltpu.SemaphoreType.DMA,
        ],
    )
    def agg_kernel(y_hbm, rows_hbm, cols_hbm, out_hbm,
                   ridx, cidx, b0, b1, b2, b3, agg_sh,
                   semi, s0, s1, s2, s3):
        bufs = (b0, b1, b2, b3)
        sems = (s0, s1, s2, s3)
        cid = lax.axis_index("c")
        sid = lax.axis_index("s")
        wid = cid * _NS + sid
        pltpu.async_copy(rows_hbm.at[wid, 0], ridx, semi)
        pltpu.async_copy(cols_hbm.at[wid, 0], cidx, semi)

        @pl.loop(0, ch)
        def _(r):
            @pl.loop(0, d, step=_L)
            def _(c0):
                b0[r, pl.ds(c0, _L)] = jnp.zeros((_L,), jnp.float32)

        @pl.loop(0, npc, step=ch)
        def _(r0):
            pltpu.sync_copy(b0, agg_sh.at[pl.ds(sid * npc + r0, ch)])

        plsc.subcore_barrier()
        pltpu.make_async_copy(rows_hbm.at[wid, 0], ridx, semi).wait()
        pltpu.make_async_copy(cols_hbm.at[wid, 0], cidx, semi).wait()

        @pl.loop(0, nwin)
        def _(w):
            for k in range(nring):
                pltpu.async_copy(y_hbm.at[ridx.at[k]], bufs[k], sems[k])

            @pl.loop(0, main_hi, step=nring)
            def _(j):
                for k in range(nring):
                    pltpu.make_async_copy(y_hbm.at[ridx.at[j + k]], bufs[k],
                                          sems[k]).wait()
                    pltpu.sync_copy(bufs[k], agg_sh.at[cidx.at[j + k]],
                                    add=True)
                    pltpu.async_copy(y_hbm.at[ridx.at[j + k + nring]],
                                     bufs[k], sems[k])

            for k in range(nring):
                pltpu.make_async_copy(y_hbm.at[ridx.at[main_hi + k]],
                                      bufs[k], sems[k]).wait()
                pltpu.sync_copy(bufs[k], agg_sh.at[cidx.at[main_hi + k]],
                                add=True)
                if k < rem:
                    pltpu.async_copy(
                        y_hbm.at[ridx.at[main_hi + nring + k]],
                        bufs[k], sems[k])
            for k in range(rem):
                pltpu.make_async_copy(y_hbm.at[ridx.at[main_hi + nring + k]],
                                      bufs[k], sems[k]).wait()
                pltpu.sync_copy(bufs[k],
                                agg_sh.at[cidx.at[main_hi + nring + k]],
                                add=True)

            @pl.when(w + 1 < nwin)
            def _():
                pltpu.sync_copy(rows_hbm.at[wid, w + 1], ridx)
                pltpu.sync_copy(cols_hbm.at[wid, w + 1], cidx)

        plsc.subcore_barrier()

        @pl.loop(0, npc, step=2 * ch)
        def _(r0):
            pltpu.sync_copy(agg_sh.at[pl.ds(sid * npc + r0, ch)], b0)
            pltpu.sync_copy(agg_sh.at[pl.ds(sid * npc + r0 + ch, ch)], b1)
            pltpu.sync_copy(b0, out_hbm.at[cid, pl.ds(sid * npc + r0, ch)])
            pltpu.sync_copy(b1,
                            out_hbm.at[cid, pl.ds(sid * npc + r0 + ch, ch)])

    return agg_kernel(y, rows4, cols4)


_BN = 2000


def _mm_body(x_ref, w_ref, o_ref):
    o_ref[...] = jnp.dot(x_ref[...], w_ref[...],
                         preferred_element_type=jnp.float32)


def _matmul(x, w):
    n, din = x.shape
    dout = w.shape[1]
    return pl.pallas_call(
        _mm_body,
        grid=(n // _BN,),
        in_specs=[pl.BlockSpec((_BN, din), lambda i: (i, 0)),
                  pl.BlockSpec((din, dout), lambda i: (0, 0))],
        out_specs=pl.BlockSpec((_BN, dout), lambda i: (i, 0)),
        out_shape=jax.ShapeDtypeStruct((n, dout), jnp.float32),
    )(x, w)


def _scale_body(t_ref, d0_ref, d1_ref, y_ref, dis_ref):
    deg = d0_ref[...] + d1_ref[...] + 1.0
    dis = lax.rsqrt(deg)
    y_ref[...] = t_ref[...] * dis
    dis_ref[...] = dis


def _scale(t, d0, d1):
    n, d = t.shape
    blk1 = pl.BlockSpec((_BN, 1), lambda i: (i, 0))
    return pl.pallas_call(
        _scale_body,
        grid=(n // _BN,),
        in_specs=[pl.BlockSpec((_BN, d), lambda i: (i, 0)), blk1, blk1],
        out_specs=[pl.BlockSpec((_BN, d), lambda i: (i, 0)), blk1],
        out_shape=[jax.ShapeDtypeStruct((n, d), jnp.float32),
                   jax.ShapeDtypeStruct((n, 1), jnp.float32)],
    )(t, d0, d1)


def _mid_body(y_ref, p0_ref, p1_ref, dis_ref, b_ref, w_ref, o_ref):
    dis = dis_ref[...]
    h = (y_ref[...] + p0_ref[0] + p1_ref[0]) * dis + b_ref[...]
    o_ref[...] = jnp.dot(h, w_ref[...],
                         preferred_element_type=jnp.float32) * dis


def _mid(y, p, dis, b, w):
    n, d = y.shape
    dout = w.shape[1]
    blk2 = pl.BlockSpec((_BN, d), lambda i: (i, 0))
    return pl.pallas_call(
        _mid_body,
        grid=(n // _BN,),
        in_specs=[blk2,
                  pl.BlockSpec((1, _BN, d), lambda i: (0, i, 0)),
                  pl.BlockSpec((1, _BN, d), lambda i: (1, i, 0)),
                  pl.BlockSpec((_BN, 1), lambda i: (i, 0)),
                  pl.BlockSpec((1, d), lambda i: (0, 0)),
                  pl.BlockSpec((d, dout), lambda i: (0, 0))],
        out_specs=pl.BlockSpec((_BN, dout), lambda i: (i, 0)),
        out_shape=jax.ShapeDtypeStruct((n, dout), jnp.float32),
    )(y, p, p, dis, b, w)


def _final_body(y_ref, q0_ref, q1_ref, dis_ref, b_ref, o_ref):
    o_ref[...] = (y_ref[...] + q0_ref[0] + q1_ref[0]) * dis_ref[...] \
        + b_ref[...]


def _final(y, q, dis, b):
    n, d = y.shape
    blk2 = pl.BlockSpec((_BN, d), lambda i: (i, 0))
    return pl.pallas_call(
        _final_body,
        grid=(n // _BN,),
        in_specs=[blk2,
                  pl.BlockSpec((1, _BN, d), lambda i: (0, i, 0)),
                  pl.BlockSpec((1, _BN, d), lambda i: (1, i, 0)),
                  pl.BlockSpec((_BN, 1), lambda i: (i, 0)),
                  pl.BlockSpec((1, d), lambda i: (0, 0))],
        out_specs=pl.BlockSpec((_BN, d), lambda i: (i, 0)),
        out_shape=jax.ShapeDtypeStruct((n, d), jnp.float32),
    )(y, q, q, dis, b)


def kernel(x, edge_index, W1, b1, W2, b2):
    n, _ = x.shape
    e = edge_index.shape[1]
    epw = e // _NW
    ch = 40
    nwin = 5
    wchunk = epw // ch // nwin
    rows4 = edge_index[0].reshape(_NW, nwin, wchunk, ch)
    cols4 = edge_index[1].reshape(_NW, nwin, wchunk, ch)
    n_pad = -(-n // (_NS * 8)) * (_NS * 8)
    n_acc = -(-n // (_NS * 128)) * (_NS * 128)

    degp = _degree_partials(
        edge_index[1].reshape(_NW, 5, epw // 80 // 5, 80),
        n_pad).reshape(_NC, n_pad)
    t1 = _matmul(x, W1)
    d0 = degp[0, :n].reshape(n, 1)
    d1 = degp[1, :n].reshape(n, 1)
    y1, dis = _scale(t1, d0, d1)
    p = _aggregate(y1, rows4, cols4, n_acc)
    y2 = _mid(y1, p, dis, b1.reshape(1, -1), W2)
    q = _aggregate(y2, rows4, cols4, n_acc)
    return _final(y2, q, dis, b2.reshape(1, -1))


# speedup vs baseline: 33.5906x; 1.0284x over previous
import functools

import jax
import jax.numpy as jnp
from jax import lax
from jax.experimental import pallas as pl
from jax.experimental.pallas import tpu as pltpu
from jax.experimental.pallas import tpu_sc as plsc

_NC = 2
_NS = 16
_L = 16
_NW = _NC * _NS

_MESH = dict(core_axis_name="c", subcore_axis_name="s")


def _degree_partials(cols4, n_pad):
    nw, nwin, wchunk, ch = cols4.shape
    zps = n_pad // _NS

    @functools.partial(
        pl.kernel,
        out_type=jax.ShapeDtypeStruct((_NC * n_pad,), jnp.float32),
        mesh=plsc.VectorSubcoreMesh(**_MESH),
        scratch_types=[
            pltpu.VMEM((wchunk, ch), jnp.int32),
            pltpu.VMEM((ch,), jnp.float32),
            pltpu.VMEM((zps,), jnp.float32),
            pltpu.VMEM_SHARED((n_pad,), jnp.float32),
            pltpu.SemaphoreType.DMA,
        ],
    )
    def deg_kernel(cols_hbm, out_hbm, cidx, ones_v, zeros_v, deg_sh, sem):
        cid = lax.axis_index("c")
        sid = lax.axis_index("s")
        wid = cid * _NS + sid

        @pl.loop(0, ch, step=_L)
        def _(i):
            ones_v[pl.ds(i, _L)] = jnp.ones((_L,), jnp.float32)

        @pl.loop(0, zps, step=_L)
        def _(i):
            zeros_v[pl.ds(i, _L)] = jnp.zeros((_L,), jnp.float32)

        pltpu.sync_copy(zeros_v, deg_sh.at[pl.ds(sid * zps, zps)])
        plsc.subcore_barrier()

        @pl.loop(0, nwin)
        def _(w):
            pltpu.sync_copy(cols_hbm.at[wid, w], cidx)

            @pl.loop(0, wchunk)
            def _(j):
                pltpu.sync_copy(ones_v, deg_sh.at[cidx.at[j]], add=True)

        plsc.subcore_barrier()
        pltpu.sync_copy(deg_sh.at[pl.ds(sid * zps, zps)], zeros_v)
        pltpu.sync_copy(zeros_v, out_hbm.at[pl.ds(cid * n_pad + sid * zps, zps)])

    return deg_kernel(cols4)


def _aggregate(y, rows4, cols4, n_acc):
    n, d = y.shape
    nw, nwin, wchunk, ch = rows4.shape
    npc = n_acc // _NS
    nring = 6
    rem = wchunk % nring
    main_hi = wchunk - nring - rem

    @functools.partial(
        pl.kernel,
        out_type=jax.ShapeDtypeStruct((_NC, n_acc, d), jnp.float32),
        mesh=plsc.VectorSubcoreMesh(**_MESH),
        scratch_types=[
            pltpu.VMEM((wchunk, ch), jnp.int32),
            pltpu.VMEM((wchunk, ch), jnp.int32),
            pltpu.VMEM((ch, d), jnp.float32),
            pltpu.VMEM((ch, d), jnp.float32),
            pltpu.VMEM((ch, d), jnp.float32),
            pltpu.VMEM((ch, d), jnp.float32),
            pltpu.VMEM((ch, d), jnp.float32),
            pltpu.VMEM((ch, d), jnp.float32),
            pltpu.VMEM_SHARED((n_acc, d), jnp.float32),
            pltpu.SemaphoreType.DMA,
            pltpu.SemaphoreType.DMA,
            pltpu.SemaphoreType.DMA,
            pltpu.SemaphoreType.DMA,
            pltpu.SemaphoreType.DMA,
            pltpu.SemaphoreType.DMA,
            pltpu.SemaphoreType.DMA,
        ],
    )
    def agg_kernel(y_hbm, rows_hbm, cols_hbm, out_hbm,
                   ridx, cidx, b0, b1, b2, b3, b4, b5, agg_sh,
                   semi, s0, s1, s2, s3, s4, s5):
        bufs = (b0, b1, b2, b3, b4, b5)
        sems = (s0, s1, s2, s3, s4, s5)
        cid = lax.axis_index("c")
        sid = lax.axis_index("s")
        wid = cid * _NS + sid
        pltpu.async_copy(rows_hbm.at[wid, 0], ridx, semi)
        pltpu.async_copy(cols_hbm.at[wid, 0], cidx, semi)

        @pl.loop(0, ch)
        def _(r):
            @pl.loop(0, d, step=_L)
            def _(c0):
                b0[r, pl.ds(c0, _L)] = jnp.zeros((_L,), jnp.float32)

        @pl.loop(0, npc, step=ch)
        def _(r0):
            pltpu.sync_copy(b0, agg_sh.at[pl.ds(sid * npc + r0, ch)])

        plsc.subcore_barrier()
        pltpu.make_async_copy(rows_hbm.at[wid, 0], ridx, semi).wait()
        pltpu.make_async_copy(cols_hbm.at[wid, 0], cidx, semi).wait()

        @pl.loop(0, nwin)
        def _(w):
            for k in range(nring):
                pltpu.async_copy(y_hbm.at[ridx.at[k]], bufs[k], sems[k])

            @pl.loop(0, main_hi, step=nring)
            def _(j):
                for k in range(nring):
                    pltpu.make_async_copy(y_hbm.at[ridx.at[j + k]], bufs[k],
                                          sems[k]).wait()
                    pltpu.sync_copy(bufs[k], agg_sh.at[cidx.at[j + k]],
                                    add=True)
                    pltpu.async_copy(y_hbm.at[ridx.at[j + k + nring]],
                                     bufs[k], sems[k])

            for k in range(nring):
                pltpu.make_async_copy(y_hbm.at[ridx.at[main_hi + k]],
                                      bufs[k], sems[k]).wait()
                pltpu.sync_copy(bufs[k], agg_sh.at[cidx.at[main_hi + k]],
                                add=True)
                if k < rem:
                    pltpu.async_copy(
                        y_hbm.at[ridx.at[main_hi + nring + k]],
                        bufs[k], sems[k])
            for k in range(rem):
                pltpu.make_async_copy(y_hbm.at[ridx.at[main_hi + nring + k]],
                                      bufs[k], sems[k]).wait()
                pltpu.sync_copy(bufs[k],
                                agg_sh.at[cidx.at[main_hi + nring + k]],
                                add=True)

            @pl.when(w + 1 < nwin)
            def _():
                pltpu.sync_copy(rows_hbm.at[wid, w + 1], ridx)
                pltpu.sync_copy(cols_hbm.at[wid, w + 1], cidx)

        plsc.subcore_barrier()

        @pl.loop(0, npc, step=2 * ch)
        def _(r0):
            pltpu.sync_copy(agg_sh.at[pl.ds(sid * npc + r0, ch)], b0)
            pltpu.sync_copy(agg_sh.at[pl.ds(sid * npc + r0 + ch, ch)], b1)
            pltpu.sync_copy(b0, out_hbm.at[cid, pl.ds(sid * npc + r0, ch)])
            pltpu.sync_copy(b1,
                            out_hbm.at[cid, pl.ds(sid * npc + r0 + ch, ch)])

    return agg_kernel(y, rows4, cols4)


_BN = 2000


def _mm_body(x_ref, w_ref, o_ref):
    o_ref[...] = jnp.dot(x_ref[...], w_ref[...],
                         preferred_element_type=jnp.float32)


def _matmul(x, w):
    n, din = x.shape
    dout = w.shape[1]
    return pl.pallas_call(
        _mm_body,
        grid=(n // _BN,),
        in_specs=[pl.BlockSpec((_BN, din), lambda i: (i, 0)),
                  pl.BlockSpec((din, dout), lambda i: (0, 0))],
        out_specs=pl.BlockSpec((_BN, dout), lambda i: (i, 0)),
        out_shape=jax.ShapeDtypeStruct((n, dout), jnp.float32),
    )(x, w)


def _scale_body(t_ref, d0_ref, d1_ref, y_ref, dis_ref):
    deg = d0_ref[...] + d1_ref[...] + 1.0
    dis = lax.rsqrt(deg)
    y_ref[...] = t_ref[...] * dis
    dis_ref[...] = dis


def _scale(t, d0, d1):
    n, d = t.shape
    blk1 = pl.BlockSpec((_BN, 1), lambda i: (i, 0))
    return pl.pallas_call(
        _scale_body,
        grid=(n // _BN,),
        in_specs=[pl.BlockSpec((_BN, d), lambda i: (i, 0)), blk1, blk1],
        out_specs=[pl.BlockSpec((_BN, d), lambda i: (i, 0)), blk1],
        out_shape=[jax.ShapeDtypeStruct((n, d), jnp.float32),
                   jax.ShapeDtypeStruct((n, 1), jnp.float32)],
    )(t, d0, d1)


def _mid_body(y_ref, p0_ref, p1_ref, dis_ref, b_ref, w_ref, o_ref):
    dis = dis_ref[...]
    h = (y_ref[...] + p0_ref[0] + p1_ref[0]) * dis + b_ref[...]
    o_ref[...] = jnp.dot(h, w_ref[...],
                         preferred_element_type=jnp.float32) * dis


def _mid(y, p, dis, b, w):
    n, d = y.shape
    dout = w.shape[1]
    blk2 = pl.BlockSpec((_BN, d), lambda i: (i, 0))
    return pl.pallas_call(
        _mid_body,
        grid=(n // _BN,),
        in_specs=[blk2,
                  pl.BlockSpec((1, _BN, d), lambda i: (0, i, 0)),
                  pl.BlockSpec((1, _BN, d), lambda i: (1, i, 0)),
                  pl.BlockSpec((_BN, 1), lambda i: (i, 0)),
                  pl.BlockSpec((1, d), lambda i: (0, 0)),
                  pl.BlockSpec((d, dout), lambda i: (0, 0))],
        out_specs=pl.BlockSpec((_BN, dout), lambda i: (i, 0)),
        out_shape=jax.ShapeDtypeStruct((n, dout), jnp.float32),
    )(y, p, p, dis, b, w)


def _final_body(y_ref, q0_ref, q1_ref, dis_ref, b_ref, o_ref):
    o_ref[...] = (y_ref[...] + q0_ref[0] + q1_ref[0]) * dis_ref[...] \
        + b_ref[...]


def _final(y, q, dis, b):
    n, d = y.shape
    blk2 = pl.BlockSpec((_BN, d), lambda i: (i, 0))
    return pl.pallas_call(
        _final_body,
        grid=(n // _BN,),
        in_specs=[blk2,
                  pl.BlockSpec((1, _BN, d), lambda i: (0, i, 0)),
                  pl.BlockSpec((1, _BN, d), lambda i: (1, i, 0)),
                  pl.BlockSpec((_BN, 1), lambda i: (i, 0)),
                  pl.BlockSpec((1, d), lambda i: (0, 0))],
        out_specs=pl.BlockSpec((_BN, d), lambda i: (i, 0)),
        out_shape=jax.ShapeDtypeStruct((n, d), jnp.float32),
    )(y, q, q, dis, b)


def kernel(x, edge_index, W1, b1, W2, b2):
    n, _ = x.shape
    e = edge_index.shape[1]
    epw = e // _NW
    ch = 40
    nwin = 5
    wchunk = epw // ch // nwin
    rows4 = edge_index[0].reshape(_NW, nwin, wchunk, ch)
    cols4 = edge_index[1].reshape(_NW, nwin, wchunk, ch)
    n_pad = -(-n // (_NS * 8)) * (_NS * 8)
    n_acc = -(-n // (_NS * 128)) * (_NS * 128)

    degp = _degree_partials(
        edge_index[1].reshape(_NW, 5, epw // 80 // 5, 80),
        n_pad).reshape(_NC, n_pad)
    t1 = _matmul(x, W1)
    d0 = degp[0, :n].reshape(n, 1)
    d1 = degp[1, :n].reshape(n, 1)
    y1, dis = _scale(t1, d0, d1)
    p = _aggregate(y1, rows4, cols4, n_acc)
    y2 = _mid(y1, p, dis, b1.reshape(1, -1), W2)
    q = _aggregate(y2, rows4, cols4, n_acc)
    return _final(y2, q, dis, b2.reshape(1, -1))


# speedup vs baseline: 34.1860x; 1.0177x over previous
import functools

import jax
import jax.numpy as jnp
from jax import lax
from jax.experimental import pallas as pl
from jax.experimental.pallas import tpu as pltpu
from jax.experimental.pallas import tpu_sc as plsc

_NC = 2
_NS = 16
_L = 16
_NW = _NC * _NS

_MESH = dict(core_axis_name="c", subcore_axis_name="s")


def _degree_partials(cols4, n_pad):
    nw, nwin, wchunk, ch = cols4.shape
    zps = n_pad // _NS

    @functools.partial(
        pl.kernel,
        out_type=jax.ShapeDtypeStruct((_NC * n_pad,), jnp.float32),
        mesh=plsc.VectorSubcoreMesh(**_MESH),
        scratch_types=[
            pltpu.VMEM((wchunk, ch), jnp.int32),
            pltpu.VMEM((ch,), jnp.float32),
            pltpu.VMEM((zps,), jnp.float32),
            pltpu.VMEM_SHARED((n_pad,), jnp.float32),
            pltpu.SemaphoreType.DMA,
        ],
    )
    def deg_kernel(cols_hbm, out_hbm, cidx, ones_v, zeros_v, deg_sh, sem):
        cid = lax.axis_index("c")
        sid = lax.axis_index("s")
        wid = cid * _NS + sid

        @pl.loop(0, ch, step=_L)
        def _(i):
            ones_v[pl.ds(i, _L)] = jnp.ones((_L,), jnp.float32)

        @pl.loop(0, zps, step=_L)
        def _(i):
            zeros_v[pl.ds(i, _L)] = jnp.zeros((_L,), jnp.float32)

        pltpu.sync_copy(zeros_v, deg_sh.at[pl.ds(sid * zps, zps)])
        plsc.subcore_barrier()

        @pl.loop(0, nwin)
        def _(w):
            pltpu.sync_copy(cols_hbm.at[wid, w], cidx)

            @pl.loop(0, wchunk)
            def _(j):
                pltpu.sync_copy(ones_v, deg_sh.at[cidx.at[j]], add=True)

        plsc.subcore_barrier()
        pltpu.sync_copy(deg_sh.at[pl.ds(sid * zps, zps)], zeros_v)
        pltpu.sync_copy(zeros_v, out_hbm.at[pl.ds(cid * n_pad + sid * zps, zps)])

    return deg_kernel(cols4)


def _aggregate(y, rows4, cols4, n_acc):
    n, d = y.shape
    nw, nwin, wchunk, ch = rows4.shape
    npc = n_acc // _NS
    nring = 6
    rem = wchunk % nring
    main_hi = wchunk - nring - rem

    @functools.partial(
        pl.kernel,
        out_type=jax.ShapeDtypeStruct((_NC, n_acc, d), jnp.float32),
        mesh=plsc.VectorSubcoreMesh(**_MESH),
        scratch_types=[
            pltpu.VMEM((wchunk, ch), jnp.int32),
            pltpu.VMEM((wchunk, ch), jnp.int32),
            pltpu.VMEM((ch, d), jnp.float32),
            pltpu.VMEM((ch, d), jnp.float32),
            pltpu.VMEM((ch, d), jnp.float32),
            pltpu.VMEM((ch, d), jnp.float32),
            pltpu.VMEM((ch, d), jnp.float32),
            pltpu.VMEM((ch, d), jnp.float32),
            pltpu.VMEM_SHARED((n_acc, d), jnp.float32),
            pltpu.SemaphoreType.DMA,
            pltpu.SemaphoreType.DMA,
            pltpu.SemaphoreType.DMA,
            pltpu.SemaphoreType.DMA,
            pltpu.SemaphoreType.DMA,
            pltpu.SemaphoreType.DMA,
            pltpu.SemaphoreType.DMA,
        ],
    )
    def agg_kernel(y_hbm, rows_hbm, cols_hbm, out_hbm,
                   ridx, cidx, b0, b1, b2, b3, b4, b5, agg_sh,
                   semi, s0, s1, s2, s3, s4, s5):
        bufs = (b0, b1, b2, b3, b4, b5)
        sems = (s0, s1, s2, s3, s4, s5)
        cid = lax.axis_index("c")
        sid = lax.axis_index("s")
        wid = cid * _NS + sid
        pltpu.async_copy(rows_hbm.at[wid, 0], ridx, semi)
        pltpu.async_copy(cols_hbm.at[wid, 0], cidx, semi)

        @pl.loop(0, ch)
        def _(r):
            @pl.loop(0, d, step=_L)
            def _(c0):
                b0[r, pl.ds(c0, _L)] = jnp.zeros((_L,), jnp.float32)

        @pl.loop(0, npc, step=ch)
        def _(r0):
            pltpu.sync_copy(b0, agg_sh.at[pl.ds(sid * npc + r0, ch)])

        plsc.subcore_barrier()
        pltpu.make_async_copy(rows_hbm.at[wid, 0], ridx, semi).wait()
        pltpu.make_async_copy(cols_hbm.at[wid, 0], cidx, semi).wait()

        @pl.loop(0, nwin)
        def _(w):
            for k in range(nring):
                pltpu.async_copy(y_hbm.at[ridx.at[k]], bufs[k], sems[k])

            @pl.loop(0, main_hi, step=nring)
            def _(j):
                for k in range(nring):
                    pltpu.make_async_copy(y_hbm.at[ridx.at[j + k]], bufs[k],
                                          sems[k]).wait()
                    pltpu.sync_copy(bufs[k], agg_sh.at[cidx.at[j + k]],
                                    add=True)
                    pltpu.async_copy(y_hbm.at[ridx.at[j + k + nring]],
                                     bufs[k], sems[k])

            for k in range(nring):
                pltpu.make_async_copy(y_hbm.at[ridx.at[main_hi + k]],
                                      bufs[k], sems[k]).wait()
                pltpu.sync_copy(bufs[k], agg_sh.at[cidx.at[main_hi + k]],
                                add=True)
                if k < rem:
                    pltpu.async_copy(
                        y_hbm.at[ridx.at[main_hi + nring + k]],
                        bufs[k], sems[k])
            for k in range(rem):
                pltpu.make_async_copy(y_hbm.at[ridx.at[main_hi + nring + k]],
                                      bufs[k], sems[k]).wait()
                pltpu.sync_copy(bufs[k],
                                agg_sh.at[cidx.at[main_hi + nring + k]],
                                add=True)

            @pl.when(w + 1 < nwin)
            def _():
                pltpu.sync_copy(rows_hbm.at[wid, w + 1], ridx)
                pltpu.sync_copy(cols_hbm.at[wid, w + 1], cidx)

        plsc.subcore_barrier()

        @pl.loop(0, npc, step=4 * ch)
        def _(r0):
            for k in range(4):
                pltpu.async_copy(
                    agg_sh.at[pl.ds(sid * npc + r0 + k * ch, ch)],
                    bufs[k], sems[k])
            for k in range(4):
                pltpu.make_async_copy(
                    agg_sh.at[pl.ds(sid * npc + r0 + k * ch, ch)],
                    bufs[k], sems[k]).wait()
                pltpu.async_copy(
                    bufs[k],
                    out_hbm.at[cid, pl.ds(sid * npc + r0 + k * ch, ch)],
                    sems[k])
            for k in range(4):
                pltpu.make_async_copy(
                    bufs[k],
                    out_hbm.at[cid, pl.ds(sid * npc + r0 + k * ch, ch)],
                    sems[k]).wait()

    return agg_kernel(y, rows4, cols4)


_BN = 2000


def _mm_scale_body(x_ref, w_ref, d0_ref, d1_ref, y_ref, dis_ref):
    deg = d0_ref[...] + d1_ref[...] + 1.0
    dis = lax.rsqrt(deg)
    y_ref[...] = jnp.dot(x_ref[...], w_ref[...],
                         preferred_element_type=jnp.float32) * dis
    dis_ref[...] = dis


def _mm_scale(x, w, d0, d1):
    n, din = x.shape
    dout = w.shape[1]
    blk1 = pl.BlockSpec((_BN, 1), lambda i: (i, 0))
    return pl.pallas_call(
        _mm_scale_body,
        grid=(n // _BN,),
        in_specs=[pl.BlockSpec((_BN, din), lambda i: (i, 0)),
                  pl.BlockSpec((din, dout), lambda i: (0, 0)),
                  blk1, blk1],
        out_specs=[pl.BlockSpec((_BN, dout), lambda i: (i, 0)), blk1],
        out_shape=[jax.ShapeDtypeStruct((n, dout), jnp.float32),
                   jax.ShapeDtypeStruct((n, 1), jnp.float32)],
    )(x, w, d0, d1)


def _mid_body(y_ref, p0_ref, p1_ref, dis_ref, b_ref, w_ref, o_ref):
    dis = dis_ref[...]
    h = (y_ref[...] + p0_ref[0] + p1_ref[0]) * dis + b_ref[...]
    o_ref[...] = jnp.dot(h, w_ref[...],
                         preferred_element_type=jnp.float32) * dis


def _mid(y, p, dis, b, w):
    n, d = y.shape
    dout = w.shape[1]
    blk2 = pl.BlockSpec((_BN, d), lambda i: (i, 0))
    return pl.pallas_call(
        _mid_body,
        grid=(n // _BN,),
        in_specs=[blk2,
                  pl.BlockSpec((1, _BN, d), lambda i: (0, i, 0)),
                  pl.BlockSpec((1, _BN, d), lambda i: (1, i, 0)),
                  pl.BlockSpec((_BN, 1), lambda i: (i, 0)),
                  pl.BlockSpec((1, d), lambda i: (0, 0)),
                  pl.BlockSpec((d, dout), lambda i: (0, 0))],
        out_specs=pl.BlockSpec((_BN, dout), lambda i: (i, 0)),
        out_shape=jax.ShapeDtypeStruct((n, dout), jnp.float32),
    )(y, p, p, dis, b, w)


def _final_body(y_ref, q0_ref, q1_ref, dis_ref, b_ref, o_ref):
    o_ref[...] = (y_ref[...] + q0_ref[0] + q1_ref[0]) * dis_ref[...] \
        + b_ref[...]


def _final(y, q, dis, b):
    n, d = y.shape
    blk2 = pl.BlockSpec((_BN, d), lambda i: (i, 0))
    return pl.pallas_call(
        _final_body,
        grid=(n // _BN,),
        in_specs=[blk2,
                  pl.BlockSpec((1, _BN, d), lambda i: (0, i, 0)),
                  pl.BlockSpec((1, _BN, d), lambda i: (1, i, 0)),
                  pl.BlockSpec((_BN, 1), lambda i: (i, 0)),
                  pl.BlockSpec((1, d), lambda i: (0, 0))],
        out_specs=pl.BlockSpec((_BN, d), lambda i: (i, 0)),
        out_shape=jax.ShapeDtypeStruct((n, d), jnp.float32),
    )(y, q, q, dis, b)


def kernel(x, edge_index, W1, b1, W2, b2):
    n, _ = x.shape
    e = edge_index.shape[1]
    epw = e // _NW
    ch = 40
    nwin = 5
    wchunk = epw // ch // nwin
    rows4 = edge_index[0].reshape(_NW, nwin, wchunk, ch)
    cols4 = edge_index[1].reshape(_NW, nwin, wchunk, ch)
    n_pad = -(-n // (_NS * 8)) * (_NS * 8)
    n_acc = -(-n // (_NS * 128)) * (_NS * 128)

    degp = _degree_partials(
        edge_index[1].reshape(_NW, 5, epw // 80 // 5, 80),
        n_pad).reshape(_NC, n_pad)
    d0 = degp[0, :n].reshape(n, 1)
    d1 = degp[1, :n].reshape(n, 1)
    y1, dis = _mm_scale(x, W1, d0, d1)
    p = _aggregate(y1, rows4, cols4, n_acc)
    y2 = _mid(y1, p, dis, b1.reshape(1, -1), W2)
    q = _aggregate(y2, rows4, cols4, n_acc)
    return _final(y2, q, dis, b2.reshape(1, -1))
